# baseline (speedup 1.0000x reference)
import functools

import jax
import jax.numpy as jnp
from jax import lax
from jax.experimental import pallas as pl
from jax.experimental.pallas import tpu as pltpu
from jax.experimental.pallas import tpu_sc as plsc

_NUM_ACTIONS = 100000
_DIM = 64
_BATCH = 16384

_NC, _NS = 2, 16
_NW = _NC * _NS
_BPW = _BATCH // _NW
_LANES = 16


def _gather_body(actions_hbm, table_hbm, outt_hbm, idx_v, rows_v, outt_v, sem):
    wid = lax.axis_index("s") * _NC + lax.axis_index("c")
    base = wid * _BPW
    pltpu.sync_copy(actions_hbm.at[pl.ds(base, _BPW)], idx_v)

    def chunk(c, carry):
        vec = idx_v[pl.ds(c * _LANES, _LANES)]
        for l in range(_LANES):
            pltpu.async_copy(
                table_hbm.at[vec[l]], rows_v.at[c * _LANES + l], sem
            )
        return carry

    lax.fori_loop(0, _BPW // _LANES, chunk, 0)

    def tchunk(c, carry):
        pltpu.make_async_copy(
            table_hbm.at[pl.ds(0, _LANES)],
            rows_v.at[pl.ds(c * _LANES, _LANES)],
            sem,
        ).wait()
        i_vec = lax.broadcasted_iota(jnp.int32, (_LANES,), 0) + c * _LANES

        def floop(f, carry2):
            f_vec = jnp.full((_LANES,), f, jnp.int32)
            vals = plsc.load_gather(rows_v, [i_vec, f_vec])
            plsc.store_scatter(outt_v, [f_vec, i_vec], vals)
            return carry2

        lax.fori_loop(0, _DIM, floop, 0)
        return carry

    lax.fori_loop(0, _BPW // _LANES, tchunk, 0)
    pltpu.sync_copy(outt_v, outt_hbm.at[:, pl.ds(base, _BPW)])


def kernel(actions, embedding_weight):
    actions = actions.astype(jnp.int32)
    mesh = plsc.VectorSubcoreMesh(core_axis_name="c", subcore_axis_name="s")
    run = pl.kernel(
        _gather_body,
        mesh=mesh,
        compiler_params=pltpu.CompilerParams(needs_layout_passes=False),
        out_type=jax.ShapeDtypeStruct((_DIM, _BATCH), jnp.float32),
        scratch_types=[
            pltpu.VMEM((_BPW,), jnp.int32),
            pltpu.VMEM((_BPW, _DIM), jnp.float32),
            pltpu.VMEM((_DIM, _BPW), jnp.float32),
            pltpu.SemaphoreType.DMA,
        ],
    )
    out_t = run(actions, embedding_weight)
    return jnp.transpose(out_t)
